# baseline (speedup 1.0000x reference)
import jax
import jax.numpy as jnp
from jax.experimental import pallas as pl
from jax.experimental.pallas import tpu as pltpu

_CHUNK_ROWS = 256
_NBUF = 10


def _pipelined_add(x_hbm, pos_hbm, out_hbm, xbuf, obuf, posbuf,
                   in_sems, out_sems, pos_sems):
    total_rows = x_hbm.shape[0]
    pos_rows = pos_hbm.shape[0]
    nchunk = total_rows // _CHUNK_ROWS
    npos = pos_rows // _CHUNK_ROWS

    def _in_copy(k):
        return pltpu.make_async_copy(
            x_hbm.at[pl.ds(k * _CHUNK_ROWS, _CHUNK_ROWS), :],
            xbuf.at[k % _NBUF],
            in_sems.at[k % _NBUF],
        )

    def _pos_copy(p):
        return pltpu.make_async_copy(
            pos_hbm.at[pl.ds(p * _CHUNK_ROWS, _CHUNK_ROWS), :],
            posbuf.at[pl.ds(p * _CHUNK_ROWS, _CHUNK_ROWS), :],
            pos_sems.at[p],
        )

    def _out_copy(k):
        return pltpu.make_async_copy(
            obuf.at[k % _NBUF],
            out_hbm.at[pl.ds(k * _CHUNK_ROWS, _CHUNK_ROWS), :],
            out_sems.at[k % _NBUF],
        )

    for k in range(min(_NBUF, nchunk)):
        if k < npos:
            _pos_copy(k).start()
        _in_copy(k).start()
    for p in range(min(_NBUF, nchunk), npos):
        _pos_copy(p).start()

    for k in range(nchunk):
        slot = k % _NBUF
        p = k % npos
        _in_copy(k).wait()
        if k < npos:
            _pos_copy(p).wait()
        if k >= _NBUF:
            _out_copy(k - _NBUF).wait()
        obuf[slot] = (
            xbuf[slot] + posbuf[pl.ds(p * _CHUNK_ROWS, _CHUNK_ROWS), :]
        )
        _out_copy(k).start()
        if k + _NBUF < nchunk:
            _in_copy(k + _NBUF).start()

    for k in range(max(nchunk - _NBUF, 0), nchunk):
        _out_copy(k).wait()


def kernel(x, pos_table):
    batch, maxlen, embed = x.shape
    x2 = x.reshape(batch * maxlen, embed)
    out = pl.pallas_call(
        _pipelined_add,
        in_specs=[
            pl.BlockSpec(memory_space=pl.ANY),
            pl.BlockSpec(memory_space=pl.ANY),
        ],
        out_specs=pl.BlockSpec(memory_space=pl.ANY),
        out_shape=jax.ShapeDtypeStruct(x2.shape, x2.dtype),
        scratch_shapes=[
            pltpu.VMEM((_NBUF, _CHUNK_ROWS, embed), jnp.float32),
            pltpu.VMEM((_NBUF, _CHUNK_ROWS, embed), jnp.float32),
            pltpu.VMEM((maxlen, embed), jnp.float32),
            pltpu.SemaphoreType.DMA((_NBUF,)),
            pltpu.SemaphoreType.DMA((_NBUF,)),
            pltpu.SemaphoreType.DMA((maxlen // _CHUNK_ROWS,)),
        ],
    )(x2, pos_table)
    return out.reshape(x.shape)


# speedup vs baseline: 1.0346x; 1.0346x over previous
import jax
import jax.numpy as jnp
from jax.experimental import pallas as pl
from jax.experimental.pallas import tpu as pltpu

_CHUNK_ROWS = 1024
_NBUF = 4


def _pipelined_add(x_hbm, pos_hbm, out_hbm, xbuf, obuf, posbuf,
                   in_sems, out_sems, pos_sems):
    total_rows = x_hbm.shape[0]
    pos_rows = pos_hbm.shape[0]
    nchunk = total_rows // _CHUNK_ROWS
    npos = pos_rows // _CHUNK_ROWS

    def _in_copy(k):
        return pltpu.make_async_copy(
            x_hbm.at[pl.ds(k * _CHUNK_ROWS, _CHUNK_ROWS), :],
            xbuf.at[k % _NBUF],
            in_sems.at[k % _NBUF],
        )

    def _pos_copy(p):
        return pltpu.make_async_copy(
            pos_hbm.at[pl.ds(p * _CHUNK_ROWS, _CHUNK_ROWS), :],
            posbuf.at[pl.ds(p * _CHUNK_ROWS, _CHUNK_ROWS), :],
            pos_sems.at[p],
        )

    def _out_copy(k):
        return pltpu.make_async_copy(
            obuf.at[k % _NBUF],
            out_hbm.at[pl.ds(k * _CHUNK_ROWS, _CHUNK_ROWS), :],
            out_sems.at[k % _NBUF],
        )

    for k in range(min(_NBUF, nchunk)):
        if k < npos:
            _pos_copy(k).start()
        _in_copy(k).start()
    for p in range(min(_NBUF, nchunk), npos):
        _pos_copy(p).start()

    for k in range(nchunk):
        slot = k % _NBUF
        p = k % npos
        _in_copy(k).wait()
        if k < npos:
            _pos_copy(p).wait()
        if k >= _NBUF:
            _out_copy(k - _NBUF).wait()
        obuf[slot] = (
            xbuf[slot] + posbuf[pl.ds(p * _CHUNK_ROWS, _CHUNK_ROWS), :]
        )
        _out_copy(k).start()
        if k + _NBUF < nchunk:
            _in_copy(k + _NBUF).start()

    for k in range(max(nchunk - _NBUF, 0), nchunk):
        _out_copy(k).wait()


def kernel(x, pos_table):
    batch, maxlen, embed = x.shape
    x2 = x.reshape(batch * maxlen, embed)
    out = pl.pallas_call(
        _pipelined_add,
        in_specs=[
            pl.BlockSpec(memory_space=pl.ANY),
            pl.BlockSpec(memory_space=pl.ANY),
        ],
        out_specs=pl.BlockSpec(memory_space=pl.ANY),
        out_shape=jax.ShapeDtypeStruct(x2.shape, x2.dtype),
        scratch_shapes=[
            pltpu.VMEM((_NBUF, _CHUNK_ROWS, embed), jnp.float32),
            pltpu.VMEM((_NBUF, _CHUNK_ROWS, embed), jnp.float32),
            pltpu.VMEM((maxlen, embed), jnp.float32),
            pltpu.SemaphoreType.DMA((_NBUF,)),
            pltpu.SemaphoreType.DMA((_NBUF,)),
            pltpu.SemaphoreType.DMA((maxlen // _CHUNK_ROWS,)),
        ],
    )(x2, pos_table)
    return out.reshape(x.shape)


# speedup vs baseline: 1.0600x; 1.0246x over previous
import jax
import jax.numpy as jnp
from jax.experimental import pallas as pl
from jax.experimental.pallas import tpu as pltpu

_CHUNK_ROWS = 1024
_NBUF = 6


def _pipelined_add(x_hbm, pos_hbm, out_hbm, xbuf, obuf, posbuf,
                   in_sems, out_sems, pos_sems):
    total_rows = x_hbm.shape[0]
    pos_rows = pos_hbm.shape[0]
    nchunk = total_rows // _CHUNK_ROWS
    npos = pos_rows // _CHUNK_ROWS

    def _in_copy(k):
        return pltpu.make_async_copy(
            x_hbm.at[pl.ds(k * _CHUNK_ROWS, _CHUNK_ROWS), :],
            xbuf.at[k % _NBUF],
            in_sems.at[k % _NBUF],
        )

    def _pos_copy(p):
        return pltpu.make_async_copy(
            pos_hbm.at[pl.ds(p * _CHUNK_ROWS, _CHUNK_ROWS), :],
            posbuf.at[pl.ds(p * _CHUNK_ROWS, _CHUNK_ROWS), :],
            pos_sems.at[p],
        )

    def _out_copy(k):
        return pltpu.make_async_copy(
            obuf.at[k % _NBUF],
            out_hbm.at[pl.ds(k * _CHUNK_ROWS, _CHUNK_ROWS), :],
            out_sems.at[k % _NBUF],
        )

    for k in range(min(_NBUF, nchunk)):
        if k < npos:
            _pos_copy(k).start()
        _in_copy(k).start()
    for p in range(min(_NBUF, nchunk), npos):
        _pos_copy(p).start()

    for k in range(nchunk):
        slot = k % _NBUF
        p = k % npos
        _in_copy(k).wait()
        if k < npos:
            _pos_copy(p).wait()
        if k >= _NBUF:
            _out_copy(k - _NBUF).wait()
        obuf[slot] = (
            xbuf[slot] + posbuf[pl.ds(p * _CHUNK_ROWS, _CHUNK_ROWS), :]
        )
        _out_copy(k).start()
        if k + _NBUF < nchunk:
            _in_copy(k + _NBUF).start()

    for k in range(max(nchunk - _NBUF, 0), nchunk):
        _out_copy(k).wait()


def kernel(x, pos_table):
    batch, maxlen, embed = x.shape
    x2 = x.reshape(batch * maxlen, embed)
    out = pl.pallas_call(
        _pipelined_add,
        in_specs=[
            pl.BlockSpec(memory_space=pl.ANY),
            pl.BlockSpec(memory_space=pl.ANY),
        ],
        out_specs=pl.BlockSpec(memory_space=pl.ANY),
        out_shape=jax.ShapeDtypeStruct(x2.shape, x2.dtype),
        scratch_shapes=[
            pltpu.VMEM((_NBUF, _CHUNK_ROWS, embed), jnp.float32),
            pltpu.VMEM((_NBUF, _CHUNK_ROWS, embed), jnp.float32),
            pltpu.VMEM((maxlen, embed), jnp.float32),
            pltpu.SemaphoreType.DMA((_NBUF,)),
            pltpu.SemaphoreType.DMA((_NBUF,)),
            pltpu.SemaphoreType.DMA((maxlen // _CHUNK_ROWS,)),
        ],
    )(x2, pos_table)
    return out.reshape(x.shape)
